# baseline (speedup 1.0000x reference)
import functools

import jax
import jax.numpy as jnp
from jax import lax
from jax.experimental import pallas as pl
from jax.experimental.pallas import tpu as pltpu
from jax.experimental.pallas import tpu_sc as plsc

N = 10000
E = 320000
H = 128
EMB = 16

NC = 2
NS = 16
NW = NC * NS
EW = E // NW
C = 80
NCH = EW // C
RPS = 632
NPAD = RPS * NS

_TE = 4000


def _gates1_body(ea_ref, pv_ref, we_ref, wc0_ref, g1_ref):
    ea = ea_ref[...]
    pv = pv_ref[...]
    ec = jnp.maximum(jnp.dot(ea, we_ref[...], preferred_element_type=jnp.float32), 0.0)
    m = (pv[:, 0:1] != 0.0).astype(jnp.float32)
    c1 = ec + jnp.dot(pv, wc0_ref[...], preferred_element_type=jnp.float32) * m
    g1_ref[...] = 1.0 / (1.0 + jnp.exp(-c1))


def _gates2_body(ea_ref, pv_ref, we_ref, wc0_ref, wc1_ref, g2_ref):
    ea = ea_ref[...]
    pv = pv_ref[...]
    ec = jnp.maximum(jnp.dot(ea, we_ref[...], preferred_element_type=jnp.float32), 0.0)
    m = (pv[:, 0:1] != 0.0).astype(jnp.float32)
    c1 = ec + jnp.dot(pv, wc0_ref[...], preferred_element_type=jnp.float32) * m
    c2 = c1 + jnp.dot(pv, wc1_ref[...], preferred_element_type=jnp.float32)
    g2_ref[...] = 1.0 / (1.0 + jnp.exp(-c2))


def _gates1(ea, pv, we, wc0):
    grid = (E // _TE,)
    return pl.pallas_call(
        _gates1_body,
        grid=grid,
        in_specs=[
            pl.BlockSpec((_TE, EMB), lambda i: (i, 0)),
            pl.BlockSpec((_TE, EMB), lambda i: (i, 0)),
            pl.BlockSpec((EMB, H), lambda i: (0, 0)),
            pl.BlockSpec((EMB, H), lambda i: (0, 0)),
        ],
        out_specs=pl.BlockSpec((_TE, H), lambda i: (i, 0)),
        out_shape=jax.ShapeDtypeStruct((E, H), jnp.float32),
        compiler_params=pltpu.CompilerParams(
            dimension_semantics=(pltpu.PARALLEL,)),
    )(ea, pv, we, wc0)


def _gates2(ea, pv, we, wc0, wc1):
    grid = (E // _TE,)
    return pl.pallas_call(
        _gates2_body,
        grid=grid,
        in_specs=[
            pl.BlockSpec((_TE, EMB), lambda i: (i, 0)),
            pl.BlockSpec((_TE, EMB), lambda i: (i, 0)),
            pl.BlockSpec((EMB, H), lambda i: (0, 0)),
            pl.BlockSpec((EMB, H), lambda i: (0, 0)),
            pl.BlockSpec((EMB, H), lambda i: (0, 0)),
        ],
        out_specs=pl.BlockSpec((_TE, H), lambda i: (i, 0)),
        out_shape=jax.ShapeDtypeStruct((E, H), jnp.float32),
        compiler_params=pltpu.CompilerParams(
            dimension_semantics=(pltpu.PARALLEL,)),
    )(ea, pv, we, wc0, wc1)


_TN = 2000


def _node_in_body(x_ref, lv_ref, wn_ref, b_ref, wl_ref, wm_ref, hpre_ref, hm_ref):
    h = jnp.maximum(
        jnp.dot(x_ref[...], wn_ref[...], preferred_element_type=jnp.float32)
        + b_ref[...], 0.0)
    hp = h + jnp.dot(lv_ref[...], wl_ref[...], preferred_element_type=jnp.float32)
    hpre_ref[...] = hp
    hm_ref[...] = jnp.dot(hp, wm_ref[...], preferred_element_type=jnp.float32)


def _node_in(x, lv, wn, b, wl, wm):
    grid = (N // _TN,)
    return pl.pallas_call(
        _node_in_body,
        grid=grid,
        in_specs=[
            pl.BlockSpec((_TN, x.shape[1]), lambda i: (i, 0)),
            pl.BlockSpec((_TN, EMB), lambda i: (i, 0)),
            pl.BlockSpec((x.shape[1], H), lambda i: (0, 0)),
            pl.BlockSpec((1, H), lambda i: (0, 0)),
            pl.BlockSpec((EMB, H), lambda i: (0, 0)),
            pl.BlockSpec((H, H), lambda i: (0, 0)),
        ],
        out_specs=[
            pl.BlockSpec((_TN, H), lambda i: (i, 0)),
            pl.BlockSpec((_TN, H), lambda i: (i, 0)),
        ],
        out_shape=[
            jax.ShapeDtypeStruct((N, H), jnp.float32),
            jax.ShapeDtypeStruct((N, H), jnp.float32),
        ],
        compiler_params=pltpu.CompilerParams(
            dimension_semantics=(pltpu.PARALLEL,)),
    )(x, lv, wn, b, wl, wm)


def _node_mid_body(hpre_ref, p0_ref, p1_ref, lv_ref, wu_ref, wl_ref, wm_ref,
                   hpre1_ref, hm1_ref):
    agg = p0_ref[...] + p1_ref[...]
    h1 = jnp.maximum(
        hpre_ref[...]
        + jnp.dot(agg, wu_ref[...], preferred_element_type=jnp.float32), 0.0)
    hp1 = h1 + jnp.dot(lv_ref[...], wl_ref[...], preferred_element_type=jnp.float32)
    hpre1_ref[...] = hp1
    hm1_ref[...] = jnp.dot(hp1, wm_ref[...], preferred_element_type=jnp.float32)


def _node_mid(hpre, p0, p1, lv, wu, wl, wm):
    grid = (N // _TN,)
    return pl.pallas_call(
        _node_mid_body,
        grid=grid,
        in_specs=[
            pl.BlockSpec((_TN, H), lambda i: (i, 0)),
            pl.BlockSpec((_TN, H), lambda i: (i, 0)),
            pl.BlockSpec((_TN, H), lambda i: (i, 0)),
            pl.BlockSpec((_TN, EMB), lambda i: (i, 0)),
            pl.BlockSpec((H, H), lambda i: (0, 0)),
            pl.BlockSpec((EMB, H), lambda i: (0, 0)),
            pl.BlockSpec((H, H), lambda i: (0, 0)),
        ],
        out_specs=[
            pl.BlockSpec((_TN, H), lambda i: (i, 0)),
            pl.BlockSpec((_TN, H), lambda i: (i, 0)),
        ],
        out_shape=[
            jax.ShapeDtypeStruct((N, H), jnp.float32),
            jax.ShapeDtypeStruct((N, H), jnp.float32),
        ],
        compiler_params=pltpu.CompilerParams(
            dimension_semantics=(pltpu.PARALLEL,)),
    )(hpre, p0, p1, lv, wu, wl, wm)


def _node_out_body(hpre_ref, p0_ref, p1_ref, wu_ref, wh_ref, out_ref):
    agg = p0_ref[...] + p1_ref[...]
    h2 = jnp.maximum(
        hpre_ref[...]
        + jnp.dot(agg, wu_ref[...], preferred_element_type=jnp.float32), 0.0)
    out_ref[...] = jnp.dot(h2, wh_ref[...], preferred_element_type=jnp.float32)


def _node_out(hpre, p0, p1, wu, wh):
    grid = (N // _TN,)
    return pl.pallas_call(
        _node_out_body,
        grid=grid,
        in_specs=[
            pl.BlockSpec((_TN, H), lambda i: (i, 0)),
            pl.BlockSpec((_TN, H), lambda i: (i, 0)),
            pl.BlockSpec((_TN, H), lambda i: (i, 0)),
            pl.BlockSpec((H, H), lambda i: (0, 0)),
            pl.BlockSpec((H, wh.shape[1]), lambda i: (0, 0)),
        ],
        out_specs=pl.BlockSpec((_TN, wh.shape[1]), lambda i: (i, 0)),
        out_shape=jax.ShapeDtypeStruct((N, wh.shape[1]), jnp.float32),
        compiler_params=pltpu.CompilerParams(
            dimension_semantics=(pltpu.PARALLEL,)),
    )(hpre, p0, p1, wu, wh)


_sc_mesh = plsc.VectorSubcoreMesh(
    core_axis_name="c", subcore_axis_name="s", num_cores=NC, num_subcores=NS)


@functools.partial(
    pl.kernel,
    out_type=jax.ShapeDtypeStruct((NC, NPAD, H), jnp.float32),
    mesh=_sc_mesh,
    scratch_types=[
        pltpu.VMEM((C,), jnp.int32),
        pltpu.VMEM((C,), jnp.int32),
        pltpu.VMEM((C,), jnp.int32),
        pltpu.VMEM((C,), jnp.int32),
        pltpu.VMEM((C, H), jnp.float32),
        pltpu.VMEM((C, H), jnp.float32),
        pltpu.VMEM((C, H), jnp.float32),
        pltpu.VMEM((C, H), jnp.float32),
        pltpu.VMEM_SHARED((NPAD, H), jnp.float32),
        pltpu.SemaphoreType.DMA,
        pltpu.SemaphoreType.DMA,
        pltpu.SemaphoreType.DMA,
        pltpu.SemaphoreType.DMA,
    ],
)
def _sc_edge(hm_hbm, g_hbm, src_hbm, dst_hbm, zero_hbm, out_hbm,
             sidx0_v, sidx1_v, didx0_v, didx1_v,
             rows0_v, rows1_v, gate0_v, gate1_v,
             acc_sh, semr0, semr1, seml0, seml1):
    c = lax.axis_index("c")
    s = lax.axis_index("s")
    wid = s * NC + c
    sidx_v = (sidx0_v, sidx1_v)
    didx_v = (didx0_v, didx1_v)
    rows_v = (rows0_v, rows1_v)
    gate_v = (gate0_v, gate1_v)
    semr = (semr0, semr1)
    seml = (seml0, seml1)

    pltpu.sync_copy(zero_hbm.at[pl.ds(s * RPS, RPS)],
                    acc_sh.at[pl.ds(s * RPS, RPS)])
    plsc.subcore_barrier()

    def issue(i, b):
        base = wid * EW + i * C
        pltpu.sync_copy(src_hbm.at[pl.ds(base, C)], sidx_v[b])
        pltpu.sync_copy(dst_hbm.at[pl.ds(base, C)], didx_v[b])
        pltpu.async_copy(hm_hbm.at[sidx_v[b]], rows_v[b], semr[b])
        pltpu.async_copy(g_hbm.at[pl.ds(base, C)], gate_v[b], seml[b])

    def drain(b):
        pltpu.make_async_copy(hm_hbm.at[pl.ds(0, C)], rows_v[b], semr[b]).wait()
        pltpu.make_async_copy(g_hbm.at[pl.ds(0, C)], gate_v[b], seml[b]).wait()

    def gate_and_scatter(i, b):
        def mul_row(r, carry2):
            for k in range(H // 16):
                sl = pl.ds(k * 16, 16)
                rows_v[b][r, sl] = rows_v[b][r, sl] * gate_v[b][r, sl]
            return carry2

        lax.fori_loop(0, C, mul_row, 0)
        pltpu.sync_copy(rows_v[b], acc_sh.at[didx_v[b]], add=True)

    issue(0, 0)

    def step(st, carry):
        for b in range(2):
            i = 2 * st + b
            issue(i + 1, 1 - b)
            drain(b)
            gate_and_scatter(i, b)
        return carry

    lax.fori_loop(0, (NCH - 1) // 2, step, 0)
    drain(0)
    gate_and_scatter(NCH - 1, 0)
    plsc.subcore_barrier()

    pltpu.sync_copy(acc_sh.at[pl.ds(s * RPS, RPS)],
                    out_hbm.at[c, pl.ds(s * RPS, RPS)])


def kernel(x, edge_index, edge_attr, loop_val, poly_val,
           W_node, b_node, W_edge_enc, W_loop, W_conn, W_msg, W_upd, W_head):
    src = edge_index[0].astype(jnp.int32)
    dst = edge_index[1].astype(jnp.int32)
    zeros = jnp.zeros((NPAD, H), jnp.float32)

    g1 = _gates1(edge_attr, poly_val, W_edge_enc, W_conn[0])

    hpre0, hm0 = _node_in(x, loop_val, W_node, b_node.reshape(1, H),
                          W_loop[0], W_msg[0])
    part0 = _sc_edge(hm0, g1, src, dst, zeros)
    g2 = _gates2(edge_attr, poly_val, W_edge_enc, W_conn[0], W_conn[1])
    hpre1, hm1 = _node_mid(hpre0, part0[0, :N], part0[1, :N], loop_val,
                           W_upd[0], W_loop[1], W_msg[1])
    part1 = _sc_edge(hm1, g2, src, dst, zeros)
    out = _node_out(hpre1, part1[0, :N], part1[1, :N], W_upd[1], W_head)
    return out


# speedup vs baseline: 1.0721x; 1.0721x over previous
import functools

import jax
import jax.numpy as jnp
from jax import lax
from jax.experimental import pallas as pl
from jax.experimental.pallas import tpu as pltpu
from jax.experimental.pallas import tpu_sc as plsc

N = 10000
E = 320000
H = 128
EMB = 16

NC = 2
NS = 16
NW = NC * NS
EW = E // NW
C = 80
NCH = EW // C
RPS = 632
NPAD = RPS * NS

_TE = 4000


def _gates1_body(ea_ref, pv_ref, we_ref, wc0_ref, g1_ref):
    ea = ea_ref[...]
    pv = pv_ref[...]
    ec = jnp.maximum(jnp.dot(ea, we_ref[...], preferred_element_type=jnp.float32), 0.0)
    m = (pv[:, 0:1] != 0.0).astype(jnp.float32)
    c1 = ec + jnp.dot(pv, wc0_ref[...], preferred_element_type=jnp.float32) * m
    g1_ref[...] = 1.0 / (1.0 + jnp.exp(-c1))


def _gates2_body(ea_ref, pv_ref, we_ref, wc0_ref, wc1_ref, g2_ref):
    ea = ea_ref[...]
    pv = pv_ref[...]
    ec = jnp.maximum(jnp.dot(ea, we_ref[...], preferred_element_type=jnp.float32), 0.0)
    m = (pv[:, 0:1] != 0.0).astype(jnp.float32)
    c1 = ec + jnp.dot(pv, wc0_ref[...], preferred_element_type=jnp.float32) * m
    c2 = c1 + jnp.dot(pv, wc1_ref[...], preferred_element_type=jnp.float32)
    g2_ref[...] = 1.0 / (1.0 + jnp.exp(-c2))


def _gates1(ea, pv, we, wc0):
    grid = (E // _TE,)
    return pl.pallas_call(
        _gates1_body,
        grid=grid,
        in_specs=[
            pl.BlockSpec((_TE, EMB), lambda i: (i, 0)),
            pl.BlockSpec((_TE, EMB), lambda i: (i, 0)),
            pl.BlockSpec((EMB, H), lambda i: (0, 0)),
            pl.BlockSpec((EMB, H), lambda i: (0, 0)),
        ],
        out_specs=pl.BlockSpec((_TE, H), lambda i: (i, 0)),
        out_shape=jax.ShapeDtypeStruct((E, H), jnp.float32),
        compiler_params=pltpu.CompilerParams(
            dimension_semantics=(pltpu.PARALLEL,)),
    )(ea, pv, we, wc0)


def _gates2(ea, pv, we, wc0, wc1):
    grid = (E // _TE,)
    return pl.pallas_call(
        _gates2_body,
        grid=grid,
        in_specs=[
            pl.BlockSpec((_TE, EMB), lambda i: (i, 0)),
            pl.BlockSpec((_TE, EMB), lambda i: (i, 0)),
            pl.BlockSpec((EMB, H), lambda i: (0, 0)),
            pl.BlockSpec((EMB, H), lambda i: (0, 0)),
            pl.BlockSpec((EMB, H), lambda i: (0, 0)),
        ],
        out_specs=pl.BlockSpec((_TE, H), lambda i: (i, 0)),
        out_shape=jax.ShapeDtypeStruct((E, H), jnp.float32),
        compiler_params=pltpu.CompilerParams(
            dimension_semantics=(pltpu.PARALLEL,)),
    )(ea, pv, we, wc0, wc1)


_TN = 2000


def _node_in_body(x_ref, lv_ref, wn_ref, b_ref, wl_ref, wm_ref, hpre_ref, hm_ref):
    h = jnp.maximum(
        jnp.dot(x_ref[...], wn_ref[...], preferred_element_type=jnp.float32)
        + b_ref[...], 0.0)
    hp = h + jnp.dot(lv_ref[...], wl_ref[...], preferred_element_type=jnp.float32)
    hpre_ref[...] = hp
    hm_ref[...] = jnp.dot(hp, wm_ref[...], preferred_element_type=jnp.float32)


def _node_in(x, lv, wn, b, wl, wm):
    grid = (N // _TN,)
    return pl.pallas_call(
        _node_in_body,
        grid=grid,
        in_specs=[
            pl.BlockSpec((_TN, x.shape[1]), lambda i: (i, 0)),
            pl.BlockSpec((_TN, EMB), lambda i: (i, 0)),
            pl.BlockSpec((x.shape[1], H), lambda i: (0, 0)),
            pl.BlockSpec((1, H), lambda i: (0, 0)),
            pl.BlockSpec((EMB, H), lambda i: (0, 0)),
            pl.BlockSpec((H, H), lambda i: (0, 0)),
        ],
        out_specs=[
            pl.BlockSpec((_TN, H), lambda i: (i, 0)),
            pl.BlockSpec((_TN, H), lambda i: (i, 0)),
        ],
        out_shape=[
            jax.ShapeDtypeStruct((N, H), jnp.float32),
            jax.ShapeDtypeStruct((N, H), jnp.float32),
        ],
        compiler_params=pltpu.CompilerParams(
            dimension_semantics=(pltpu.PARALLEL,)),
    )(x, lv, wn, b, wl, wm)


def _node_mid_body(hpre_ref, p0_ref, p1_ref, lv_ref, wu_ref, wl_ref, wm_ref,
                   hpre1_ref, hm1_ref):
    agg = p0_ref[...] + p1_ref[...]
    h1 = jnp.maximum(
        hpre_ref[...]
        + jnp.dot(agg, wu_ref[...], preferred_element_type=jnp.float32), 0.0)
    hp1 = h1 + jnp.dot(lv_ref[...], wl_ref[...], preferred_element_type=jnp.float32)
    hpre1_ref[...] = hp1
    hm1_ref[...] = jnp.dot(hp1, wm_ref[...], preferred_element_type=jnp.float32)


def _node_mid(hpre, p0, p1, lv, wu, wl, wm):
    grid = (N // _TN,)
    return pl.pallas_call(
        _node_mid_body,
        grid=grid,
        in_specs=[
            pl.BlockSpec((_TN, H), lambda i: (i, 0)),
            pl.BlockSpec((_TN, H), lambda i: (i, 0)),
            pl.BlockSpec((_TN, H), lambda i: (i, 0)),
            pl.BlockSpec((_TN, EMB), lambda i: (i, 0)),
            pl.BlockSpec((H, H), lambda i: (0, 0)),
            pl.BlockSpec((EMB, H), lambda i: (0, 0)),
            pl.BlockSpec((H, H), lambda i: (0, 0)),
        ],
        out_specs=[
            pl.BlockSpec((_TN, H), lambda i: (i, 0)),
            pl.BlockSpec((_TN, H), lambda i: (i, 0)),
        ],
        out_shape=[
            jax.ShapeDtypeStruct((N, H), jnp.float32),
            jax.ShapeDtypeStruct((N, H), jnp.float32),
        ],
        compiler_params=pltpu.CompilerParams(
            dimension_semantics=(pltpu.PARALLEL,)),
    )(hpre, p0, p1, lv, wu, wl, wm)


def _node_out_body(hpre_ref, p0_ref, p1_ref, wu_ref, wh_ref, out_ref):
    agg = p0_ref[...] + p1_ref[...]
    h2 = jnp.maximum(
        hpre_ref[...]
        + jnp.dot(agg, wu_ref[...], preferred_element_type=jnp.float32), 0.0)
    out_ref[...] = jnp.dot(h2, wh_ref[...], preferred_element_type=jnp.float32)


def _node_out(hpre, p0, p1, wu, wh):
    grid = (N // _TN,)
    return pl.pallas_call(
        _node_out_body,
        grid=grid,
        in_specs=[
            pl.BlockSpec((_TN, H), lambda i: (i, 0)),
            pl.BlockSpec((_TN, H), lambda i: (i, 0)),
            pl.BlockSpec((_TN, H), lambda i: (i, 0)),
            pl.BlockSpec((H, H), lambda i: (0, 0)),
            pl.BlockSpec((H, wh.shape[1]), lambda i: (0, 0)),
        ],
        out_specs=pl.BlockSpec((_TN, wh.shape[1]), lambda i: (i, 0)),
        out_shape=jax.ShapeDtypeStruct((N, wh.shape[1]), jnp.float32),
        compiler_params=pltpu.CompilerParams(
            dimension_semantics=(pltpu.PARALLEL,)),
    )(hpre, p0, p1, wu, wh)


_sc_mesh = plsc.VectorSubcoreMesh(
    core_axis_name="c", subcore_axis_name="s", num_cores=NC, num_subcores=NS)


@functools.partial(
    pl.kernel,
    out_type=jax.ShapeDtypeStruct((NC, NPAD, H), jnp.float32),
    mesh=_sc_mesh,
    scratch_types=[
        pltpu.VMEM((2 * C,), jnp.int32),
        pltpu.VMEM((2 * C,), jnp.int32),
        pltpu.VMEM((C,), jnp.int32),
        pltpu.VMEM((C, H), jnp.float32),
        pltpu.VMEM((C, H), jnp.float32),
        pltpu.VMEM((C, H), jnp.float32),
        pltpu.VMEM((C, H), jnp.float32),
        pltpu.VMEM_SHARED((NPAD, H), jnp.float32),
        pltpu.SemaphoreType.DMA,
        pltpu.SemaphoreType.DMA,
        pltpu.SemaphoreType.DMA,
        pltpu.SemaphoreType.DMA,
        pltpu.SemaphoreType.DMA,
        pltpu.SemaphoreType.DMA,
    ],
)
def _sc_edge(hm_hbm, g_hbm, idx_hbm, zero_hbm, out_hbm,
             ibuf0_v, ibuf1_v, dchunk_v,
             rows0_v, rows1_v, gate0_v, gate1_v,
             acc_sh, semr0, semr1, seml0, seml1, semi0, semi1):
    c = lax.axis_index("c")
    s = lax.axis_index("s")
    wid = s * NC + c
    ibuf = (ibuf0_v, ibuf1_v)
    rows_v = (rows0_v, rows1_v)
    gate_v = (gate0_v, gate1_v)
    semr = (semr0, semr1)
    seml = (seml0, seml1)
    semi = (semi0, semi1)

    pltpu.sync_copy(zero_hbm.at[pl.ds(s * RPS, RPS)],
                    acc_sh.at[pl.ds(s * RPS, RPS)])
    plsc.subcore_barrier()

    def issue_idx(i, b):
        pltpu.async_copy(idx_hbm.at[pl.ds((wid * NCH + i) * 2 * C, 2 * C)],
                         ibuf[b], semi[b])

    def wait_idx(b):
        pltpu.make_async_copy(idx_hbm.at[pl.ds(0, 2 * C)], ibuf[b],
                              semi[b]).wait()

    def issue_data(i, b):
        pltpu.async_copy(hm_hbm.at[ibuf[b].at[pl.ds(0, C)]],
                         rows_v[b], semr[b])
        pltpu.async_copy(g_hbm.at[pl.ds(wid * EW + i * C, C)],
                         gate_v[b], seml[b])

    def drain_data(b):
        pltpu.make_async_copy(hm_hbm.at[pl.ds(0, C)], rows_v[b], semr[b]).wait()
        pltpu.make_async_copy(g_hbm.at[pl.ds(0, C)], gate_v[b], seml[b]).wait()

    def gate_and_scatter(b):
        def mul_row(r, carry2):
            for k in range(H // 16):
                sl = pl.ds(k * 16, 16)
                rows_v[b][r, sl] = rows_v[b][r, sl] * gate_v[b][r, sl]
            return carry2

        lax.fori_loop(0, C, mul_row, 0)
        for k in range(C // 16):
            dchunk_v[pl.ds(k * 16, 16)] = ibuf[b][pl.ds(C + k * 16, 16)]
        pltpu.sync_copy(rows_v[b], acc_sh.at[dchunk_v], add=True)

    issue_idx(0, 0)
    wait_idx(0)
    issue_data(0, 0)
    issue_idx(1, 1)

    def step(st, carry):
        for b in range(2):
            i = 2 * st + b
            wait_idx(1 - b)
            issue_data(i + 1, 1 - b)
            drain_data(b)
            gate_and_scatter(b)

            @pl.when(i + 2 < NCH)
            def _():
                issue_idx(i + 2, b)
        return carry

    lax.fori_loop(0, (NCH - 1) // 2, step, 0)
    drain_data(0)
    gate_and_scatter(0)
    plsc.subcore_barrier()

    pltpu.sync_copy(acc_sh.at[pl.ds(s * RPS, RPS)],
                    out_hbm.at[c, pl.ds(s * RPS, RPS)])


def kernel(x, edge_index, edge_attr, loop_val, poly_val,
           W_node, b_node, W_edge_enc, W_loop, W_conn, W_msg, W_upd, W_head):
    src = edge_index[0].astype(jnp.int32)
    dst = edge_index[1].astype(jnp.int32)
    idx2 = jnp.stack([src.reshape(-1, C), dst.reshape(-1, C)],
                     axis=1).reshape(-1)
    zeros = jnp.zeros((NPAD, H), jnp.float32)

    g1 = _gates1(edge_attr, poly_val, W_edge_enc, W_conn[0])

    hpre0, hm0 = _node_in(x, loop_val, W_node, b_node.reshape(1, H),
                          W_loop[0], W_msg[0])
    part0 = _sc_edge(hm0, g1, idx2, zeros)
    g2 = _gates2(edge_attr, poly_val, W_edge_enc, W_conn[0], W_conn[1])
    hpre1, hm1 = _node_mid(hpre0, part0[0, :N], part0[1, :N], loop_val,
                           W_upd[0], W_loop[1], W_msg[1])
    part1 = _sc_edge(hm1, g2, idx2, zeros)
    out = _node_out(hpre1, part1[0, :N], part1[1, :N], W_upd[1], W_head)
    return out
